# baseline (speedup 1.0000x reference)
import functools

import jax
import jax.numpy as jnp
from jax import lax
from jax.experimental import pallas as pl
from jax.experimental.pallas import tpu as pltpu
from jax.experimental.pallas import tpu_sc as plsc

_N = 10000
_E = 320000
_D = 128
_G = 64

_NC = 2
_NS = 16
_NW = _NC * _NS
_EW = _E // _NW
_KE = 125
_NCH = _EW // _KE
_SEG = 16
_SEGN = _NCH // _SEG
_RZ = 104
_WT = 624
_WTAIL = _N - _NS * _WT

_R = 400
_NB = _N // _R


def _sc_mesh():
    return plsc.VectorSubcoreMesh(core_axis_name="c", subcore_axis_name="s")


def _sc_deg(dst):
    @functools.partial(
        pl.kernel,
        out_type=jax.ShapeDtypeStruct((_NC, _N), jnp.float32),
        mesh=_sc_mesh(),
        scratch_types=[
            pltpu.VMEM((_NCH, _KE), jnp.int32),
            pltpu.VMEM((_KE,), jnp.float32),
            pltpu.VMEM((_N,), jnp.float32),
            pltpu.SemaphoreType.DMA,
            pltpu.VMEM_SHARED((_N,), jnp.float32),
        ],
    )
    def deg_kernel(dst_hbm, out_hbm, didx_v, ones_v, zero_v, sem, deg_sh):
        c = lax.axis_index("c")
        s = lax.axis_index("s")
        wid = c * _NS + s

        pltpu.sync_copy(dst_hbm.at[wid], didx_v)

        def fill_ones(i, carry):
            ones_v[pl.ds(i * 16, 16)] = jnp.ones((16,), jnp.float32)
            return carry

        lax.fori_loop(0, _KE // 16, fill_ones, 0)
        ones_v[pl.ds(_KE - 16, 16)] = jnp.ones((16,), jnp.float32)

        @pl.when(s == 0)
        def _():
            def fill_zero(i, carry):
                zero_v[pl.ds(i * 16, 16)] = jnp.zeros((16,), jnp.float32)
                return carry

            lax.fori_loop(0, _N // 16, fill_zero, 0)
            pltpu.sync_copy(zero_v, deg_sh)

        plsc.subcore_barrier()

        _GRP = 5

        def grp(g, carry):
            for t in range(_GRP):
                pltpu.async_copy(ones_v, deg_sh.at[didx_v.at[g * _GRP + t]],
                                 sem, add=True)
            for t in range(_GRP):
                pltpu.make_async_copy(ones_v, deg_sh.at[didx_v.at[0]],
                                      sem).wait()
            return carry

        lax.fori_loop(0, _NCH // _GRP, grp, 0)
        plsc.subcore_barrier()

        @pl.when(s == 0)
        def _():
            pltpu.sync_copy(deg_sh, out_hbm.at[c])

    return deg_kernel(dst)


def _sc_edge_agg(hs, src, dst):
    @functools.partial(
        pl.kernel,
        out_type=jax.ShapeDtypeStruct((_NC, _N, _D), jnp.float32),
        mesh=_sc_mesh(),
        scratch_types=[
            pltpu.VMEM((_SEG, _KE), jnp.int32),
            pltpu.VMEM((_SEG, _KE), jnp.int32),
            pltpu.VMEM((_KE, _D), jnp.float32),
            pltpu.VMEM((_KE, _D), jnp.float32),
            pltpu.SemaphoreType.DMA,
            pltpu.SemaphoreType.DMA,
            pltpu.SemaphoreType.DMA,
            pltpu.SemaphoreType.DMA,
            pltpu.VMEM_SHARED((_N, _D), jnp.float32),
        ],
    )
    def agg_kernel(hs_hbm, src_hbm, dst_hbm, out_hbm, sidx_v, didx_v,
                   rows0_v, rows1_v, sem0, sem1, semS0, semS1, agg_sh):
        c = lax.axis_index("c")
        s = lax.axis_index("s")
        wid = c * _NS + s

        def zrow(i, carry):
            for j in range(_D // 16):
                rows0_v[i, pl.ds(j * 16, 16)] = jnp.zeros((16,), jnp.float32)
            return carry

        lax.fori_loop(0, _KE, zrow, 0)

        def zchunk(k, carry):
            pltpu.sync_copy(rows0_v.at[pl.ds(0, _RZ)],
                            agg_sh.at[pl.ds(s * _WT + k * _RZ, _RZ)])
            return carry

        lax.fori_loop(0, _WT // _RZ, zchunk, 0)

        @pl.when(s == 0)
        def _():
            pltpu.sync_copy(rows0_v.at[pl.ds(0, _WTAIL)],
                            agg_sh.at[pl.ds(_NS * _WT, _WTAIL)])

        plsc.subcore_barrier()

        def wait_g(rows_v, sem):
            pltpu.make_async_copy(hs_hbm.at[sidx_v.at[0]], rows_v, sem).wait()

        def wait_s(rows_v, sem):
            pltpu.make_async_copy(rows_v, agg_sh.at[didx_v.at[0]],
                                  sem).wait()

        def seg_body(sg, carry):
            pltpu.sync_copy(src_hbm.at[wid, sg], sidx_v)
            pltpu.sync_copy(dst_hbm.at[wid, sg], didx_v)
            pltpu.async_copy(hs_hbm.at[sidx_v.at[0]], rows0_v, sem0)
            pltpu.async_copy(hs_hbm.at[sidx_v.at[1]], rows1_v, sem1)

            def step(g, carry2):
                j0 = 2 * g
                wait_g(rows0_v, sem0)
                pltpu.async_copy(rows0_v, agg_sh.at[didx_v.at[j0]], semS0,
                                 add=True)
                wait_g(rows1_v, sem1)
                pltpu.async_copy(rows1_v, agg_sh.at[didx_v.at[j0 + 1]],
                                 semS1, add=True)
                wait_s(rows0_v, semS0)
                pltpu.async_copy(hs_hbm.at[sidx_v.at[j0 + 2]], rows0_v, sem0)
                wait_s(rows1_v, semS1)
                pltpu.async_copy(hs_hbm.at[sidx_v.at[j0 + 3]], rows1_v, sem1)
                return carry2

            lax.fori_loop(0, (_SEG - 2) // 2, step, 0)
            wait_g(rows0_v, sem0)
            pltpu.async_copy(rows0_v, agg_sh.at[didx_v.at[_SEG - 2]], semS0,
                             add=True)
            wait_g(rows1_v, sem1)
            pltpu.async_copy(rows1_v, agg_sh.at[didx_v.at[_SEG - 1]], semS1,
                             add=True)
            wait_s(rows0_v, semS0)
            wait_s(rows1_v, semS1)
            return carry

        lax.fori_loop(0, _SEGN, seg_body, 0)
        plsc.subcore_barrier()
        pltpu.sync_copy(agg_sh.at[pl.ds(s * _WT, _WT)],
                        out_hbm.at[c, pl.ds(s * _WT, _WT)])

        @pl.when(s == 0)
        def _():
            pltpu.sync_copy(agg_sh.at[pl.ds(_NS * _WT, _WTAIL)],
                            out_hbm.at[c, pl.ds(_NS * _WT, _WTAIL)])

    return agg_kernel(hs, src, dst)


def _colvec(row, n):
    ii = lax.broadcasted_iota(jnp.int32, (n, n), 0)
    jj = lax.broadcasted_iota(jnp.int32, (n, n), 1)
    eye = jnp.where(ii == jj, jnp.float32(1.0), jnp.float32(0.0))
    return lax.dot_general(eye, row, (((1,), (1,)), ((), ())),
                           preferred_element_type=jnp.float32)


def _dinv_block(dega_ref, degb_ref):
    deg_row = dega_ref[0] + degb_ref[0] + 1.0
    return lax.rsqrt(_colvec(deg_row, _R))


def _tc_scale_matmul(x, W0, dega, degb):
    def body(x_ref, w_ref, dega_ref, degb_ref, out_ref):
        dinv = _dinv_block(dega_ref, degb_ref)
        out_ref[...] = jnp.dot(x_ref[...], w_ref[...],
                               preferred_element_type=jnp.float32) * dinv

    return pl.pallas_call(
        body,
        grid=(_NB,),
        in_specs=[
            pl.BlockSpec((_R, _D), lambda i: (i, 0)),
            pl.BlockSpec((_D, _D), lambda i: (0, 0)),
            pl.BlockSpec((1, 1, _R), lambda i: (i, 0, 0)),
            pl.BlockSpec((1, 1, _R), lambda i: (i, 0, 0)),
        ],
        out_specs=pl.BlockSpec((_R, _D), lambda i: (i, 0)),
        out_shape=jax.ShapeDtypeStruct((_N, _D), jnp.float32),
    )(x, W0, dega, degb)


def _ln_relu(z, g_ref, be_ref):
    mu = jnp.mean(z, axis=-1, keepdims=True)
    d = z - mu
    var = jnp.mean(d * d, axis=-1, keepdims=True)
    zn = d * lax.rsqrt(var + 1e-5) * g_ref[...] + be_ref[...]
    return jnp.maximum(zn, 0.0)


def _tc_layer0(agg_a, agg_b, hs0, dega, degb, b0, g0, be0, W1):
    def body(a_ref, b_ref, hs_ref, dega_ref, degb_ref, bias_ref, g_ref,
             be_ref, w_ref, h0_ref, hs1_ref):
        dinv = _dinv_block(dega_ref, degb_ref)
        z = (a_ref[...] + b_ref[...] + hs_ref[...]) * dinv + bias_ref[...]
        h0 = _ln_relu(z, g_ref, be_ref)
        h0_ref[...] = h0
        hs1_ref[...] = jnp.dot(h0, w_ref[...],
                               preferred_element_type=jnp.float32) * dinv

    return pl.pallas_call(
        body,
        grid=(_NB,),
        in_specs=[
            pl.BlockSpec((_R, _D), lambda i: (i, 0)),
            pl.BlockSpec((_R, _D), lambda i: (i, 0)),
            pl.BlockSpec((_R, _D), lambda i: (i, 0)),
            pl.BlockSpec((1, 1, _R), lambda i: (i, 0, 0)),
            pl.BlockSpec((1, 1, _R), lambda i: (i, 0, 0)),
            pl.BlockSpec((1, _D), lambda i: (0, 0)),
            pl.BlockSpec((1, _D), lambda i: (0, 0)),
            pl.BlockSpec((1, _D), lambda i: (0, 0)),
            pl.BlockSpec((_D, _D), lambda i: (0, 0)),
        ],
        out_specs=[
            pl.BlockSpec((_R, _D), lambda i: (i, 0)),
            pl.BlockSpec((_R, _D), lambda i: (i, 0)),
        ],
        out_shape=[
            jax.ShapeDtypeStruct((_N, _D), jnp.float32),
            jax.ShapeDtypeStruct((_N, _D), jnp.float32),
        ],
    )(agg_a, agg_b, hs0, dega, degb, b0, g0, be0, W1)


def _tc_layer1_pool(agg_a, agg_b, hs1, h0, dega, degb, b1, g1, be1, batch3):
    def body(a_ref, b_ref, hs_ref, h0_ref, dega_ref, degb_ref, bias_ref,
             g_ref, be_ref, bt_ref, psum_ref, pcnt_ref):
        dinv = _dinv_block(dega_ref, degb_ref)
        z = (a_ref[...] + b_ref[...] + hs_ref[...]) * dinv + bias_ref[...]
        h1 = _ln_relu(z, g_ref, be_ref) + h0_ref[...]
        bt = bt_ref[0]
        gi = lax.broadcasted_iota(jnp.int32, (_G, _R), 0)
        oh = jnp.where(gi == bt, jnp.float32(1.0), jnp.float32(0.0))
        ps = jnp.dot(oh, h1, preferred_element_type=jnp.float32)
        cnt = jnp.sum(oh, axis=1, keepdims=True) * jnp.ones(
            (1, _D), jnp.float32)

        @pl.when(pl.program_id(0) == 0)
        def _():
            psum_ref[...] = ps
            pcnt_ref[...] = cnt

        @pl.when(pl.program_id(0) != 0)
        def _():
            psum_ref[...] += ps
            pcnt_ref[...] += cnt

    return pl.pallas_call(
        body,
        grid=(_NB,),
        in_specs=[
            pl.BlockSpec((_R, _D), lambda i: (i, 0)),
            pl.BlockSpec((_R, _D), lambda i: (i, 0)),
            pl.BlockSpec((_R, _D), lambda i: (i, 0)),
            pl.BlockSpec((_R, _D), lambda i: (i, 0)),
            pl.BlockSpec((1, 1, _R), lambda i: (i, 0, 0)),
            pl.BlockSpec((1, 1, _R), lambda i: (i, 0, 0)),
            pl.BlockSpec((1, _D), lambda i: (0, 0)),
            pl.BlockSpec((1, _D), lambda i: (0, 0)),
            pl.BlockSpec((1, _D), lambda i: (0, 0)),
            pl.BlockSpec((1, 1, _R), lambda i: (i, 0, 0)),
        ],
        out_specs=[
            pl.BlockSpec((_G, _D), lambda i: (0, 0)),
            pl.BlockSpec((_G, _D), lambda i: (0, 0)),
        ],
        out_shape=[
            jax.ShapeDtypeStruct((_G, _D), jnp.float32),
            jax.ShapeDtypeStruct((_G, _D), jnp.float32),
        ],
    )(agg_a, agg_b, hs1, h0, dega, degb, b1, g1, be1, batch3)


def _tc_head(psum, pcnt, fW1, fb1, fW2, fb2, fW3, fb3):
    def body(ps_ref, pc_ref, w1_ref, b1_ref, w2_ref, b2_ref, w3_ref, b3_ref,
             out_ref):
        pooled = ps_ref[...] / jnp.maximum(pc_ref[...], 1.0)
        z = jnp.maximum(jnp.dot(pooled, w1_ref[...],
                                preferred_element_type=jnp.float32)
                        + b1_ref[...], 0.0)
        z = jnp.maximum(jnp.dot(z, w2_ref[...],
                                preferred_element_type=jnp.float32)
                        + b2_ref[...], 0.0)
        z = jnp.dot(z, w3_ref[...],
                    preferred_element_type=jnp.float32) + b3_ref[...]
        out_ref[...] = jnp.tanh(z)

    return pl.pallas_call(
        body,
        out_shape=jax.ShapeDtypeStruct((_G, _D), jnp.float32),
    )(psum, pcnt, fW1, fb1, fW2, fb2, fW3, fb3)


def kernel(x, edge_index, batch, W0, b0, g0, be0, W1, b1, g1, be1,
           fW1, fb1, fW2, fb2, fW3, fb3):
    src = edge_index[0].astype(jnp.int32).reshape(_NW, _SEGN, _SEG, _KE)
    dst = edge_index[1].astype(jnp.int32).reshape(_NW, _SEGN, _SEG, _KE)
    dst3 = dst.reshape(_NW, _NCH, _KE)

    degp = _sc_deg(dst3)
    dega = degp[0].reshape(_NB, 1, _R)
    degb = degp[1].reshape(_NB, 1, _R)

    hs0 = _tc_scale_matmul(x, W0, dega, degb)
    aggp0 = _sc_edge_agg(hs0, src, dst)
    h0, hs1 = _tc_layer0(aggp0[0], aggp0[1], hs0, dega, degb,
                         b0.reshape(1, _D), g0.reshape(1, _D),
                         be0.reshape(1, _D), W1)

    aggp1 = _sc_edge_agg(hs1, src, dst)
    batch3 = batch.astype(jnp.int32).reshape(_NB, 1, _R)
    psum, pcnt = _tc_layer1_pool(aggp1[0], aggp1[1], hs1, h0, dega, degb,
                                 b1.reshape(1, _D), g1.reshape(1, _D),
                                 be1.reshape(1, _D), batch3)

    return _tc_head(psum, pcnt,
                    fW1, fb1.reshape(1, -1),
                    fW2, fb2.reshape(1, -1),
                    fW3, fb3.reshape(1, -1))


# speedup vs baseline: 1.1866x; 1.1866x over previous
import functools

import jax
import jax.numpy as jnp
from jax import lax
from jax.experimental import pallas as pl
from jax.experimental.pallas import tpu as pltpu
from jax.experimental.pallas import tpu_sc as plsc

_N = 10000
_E = 320000
_D = 128
_G = 64

_NC = 2
_NS = 16
_NW = _NC * _NS
_EW = _E // _NW
_KE = 125
_NCH = _EW // _KE
_SEG = 16
_SEGN = _NCH // _SEG
_RZ = 104
_WT = 624
_WTAIL = _N - _NS * _WT

_R = 400
_NB = _N // _R


def _sc_mesh():
    return plsc.VectorSubcoreMesh(core_axis_name="c", subcore_axis_name="s")


def _sc_deg(dst):
    @functools.partial(
        pl.kernel,
        out_type=jax.ShapeDtypeStruct((_NC, _N), jnp.float32),
        mesh=_sc_mesh(),
        scratch_types=[
            pltpu.VMEM((_NCH, _KE), jnp.int32),
            pltpu.VMEM((_KE,), jnp.float32),
            pltpu.VMEM((_N,), jnp.float32),
            pltpu.SemaphoreType.DMA,
            pltpu.VMEM_SHARED((_N,), jnp.float32),
        ],
    )
    def deg_kernel(dst_hbm, out_hbm, didx_v, ones_v, zero_v, sem, deg_sh):
        c = lax.axis_index("c")
        s = lax.axis_index("s")
        wid = c * _NS + s

        pltpu.sync_copy(dst_hbm.at[wid], didx_v)

        def fill_ones(i, carry):
            ones_v[pl.ds(i * 16, 16)] = jnp.ones((16,), jnp.float32)
            return carry

        lax.fori_loop(0, _KE // 16, fill_ones, 0)
        ones_v[pl.ds(_KE - 16, 16)] = jnp.ones((16,), jnp.float32)

        @pl.when(s == 0)
        def _():
            def fill_zero(i, carry):
                zero_v[pl.ds(i * 16, 16)] = jnp.zeros((16,), jnp.float32)
                return carry

            lax.fori_loop(0, _N // 16, fill_zero, 0)
            pltpu.sync_copy(zero_v, deg_sh)

        plsc.subcore_barrier()

        _GRP = 5

        def grp(g, carry):
            for t in range(_GRP):
                pltpu.async_copy(ones_v, deg_sh.at[didx_v.at[g * _GRP + t]],
                                 sem, add=True)
            for t in range(_GRP):
                pltpu.make_async_copy(ones_v, deg_sh.at[didx_v.at[0]],
                                      sem).wait()
            return carry

        lax.fori_loop(0, _NCH // _GRP, grp, 0)
        plsc.subcore_barrier()

        @pl.when(s == 0)
        def _():
            pltpu.sync_copy(deg_sh, out_hbm.at[c])

    return deg_kernel(dst)


def _sc_edge_agg(hs, src, dst):
    @functools.partial(
        pl.kernel,
        out_type=jax.ShapeDtypeStruct((_NC, _N, _D), jnp.float32),
        mesh=_sc_mesh(),
        scratch_types=[
            pltpu.VMEM((_SEG, _KE), jnp.int32),
            pltpu.VMEM((_SEG, _KE), jnp.int32),
            pltpu.VMEM((_KE, _D), jnp.float32),
            pltpu.VMEM((_KE, _D), jnp.float32),
            pltpu.SemaphoreType.DMA,
            pltpu.SemaphoreType.DMA,
            pltpu.VMEM_SHARED((_N, _D), jnp.float32),
        ],
    )
    def agg_kernel(hs_hbm, src_hbm, dst_hbm, out_hbm, sidx_v, didx_v,
                   rows0_v, rows1_v, sem0, sem1, agg_sh):
        c = lax.axis_index("c")
        s = lax.axis_index("s")
        wid = c * _NS + s

        def zrow(i, carry):
            for j in range(_D // 16):
                rows0_v[i, pl.ds(j * 16, 16)] = jnp.zeros((16,), jnp.float32)
            return carry

        lax.fori_loop(0, _KE, zrow, 0)

        def zchunk(k, carry):
            pltpu.sync_copy(rows0_v.at[pl.ds(0, _RZ)],
                            agg_sh.at[pl.ds(s * _WT + k * _RZ, _RZ)])
            return carry

        lax.fori_loop(0, _WT // _RZ, zchunk, 0)

        @pl.when(s == 0)
        def _():
            pltpu.sync_copy(rows0_v.at[pl.ds(0, _WTAIL)],
                            agg_sh.at[pl.ds(_NS * _WT, _WTAIL)])

        plsc.subcore_barrier()

        def seg_body(sg, carry):
            pltpu.sync_copy(src_hbm.at[wid, sg], sidx_v)
            pltpu.sync_copy(dst_hbm.at[wid, sg], didx_v)
            pltpu.async_copy(hs_hbm.at[sidx_v.at[0]], rows0_v, sem0)

            def step(g, carry2):
                j0 = 2 * g
                pltpu.async_copy(hs_hbm.at[sidx_v.at[j0 + 1]], rows1_v, sem1)
                pltpu.make_async_copy(hs_hbm.at[sidx_v.at[0]], rows0_v,
                                      sem0).wait()
                pltpu.sync_copy(rows0_v, agg_sh.at[didx_v.at[j0]], add=True)
                pltpu.async_copy(hs_hbm.at[sidx_v.at[j0 + 2]], rows0_v, sem0)
                pltpu.make_async_copy(hs_hbm.at[sidx_v.at[0]], rows1_v,
                                      sem1).wait()
                pltpu.sync_copy(rows1_v, agg_sh.at[didx_v.at[j0 + 1]],
                                add=True)
                return carry2

            lax.fori_loop(0, (_SEG - 2) // 2, step, 0)
            pltpu.async_copy(hs_hbm.at[sidx_v.at[_SEG - 1]], rows1_v, sem1)
            pltpu.make_async_copy(hs_hbm.at[sidx_v.at[0]], rows0_v,
                                  sem0).wait()
            pltpu.sync_copy(rows0_v, agg_sh.at[didx_v.at[_SEG - 2]], add=True)
            pltpu.make_async_copy(hs_hbm.at[sidx_v.at[0]], rows1_v,
                                  sem1).wait()
            pltpu.sync_copy(rows1_v, agg_sh.at[didx_v.at[_SEG - 1]], add=True)
            return carry

        lax.fori_loop(0, _SEGN, seg_body, 0)
        plsc.subcore_barrier()
        pltpu.sync_copy(agg_sh.at[pl.ds(s * _WT, _WT)],
                        out_hbm.at[c, pl.ds(s * _WT, _WT)])

        @pl.when(s == 0)
        def _():
            pltpu.sync_copy(agg_sh.at[pl.ds(_NS * _WT, _WTAIL)],
                            out_hbm.at[c, pl.ds(_NS * _WT, _WTAIL)])

    return agg_kernel(hs, src, dst)


def _colvec(row, n):
    ii = lax.broadcasted_iota(jnp.int32, (n, n), 0)
    jj = lax.broadcasted_iota(jnp.int32, (n, n), 1)
    eye = jnp.where(ii == jj, jnp.float32(1.0), jnp.float32(0.0))
    return lax.dot_general(eye, row, (((1,), (1,)), ((), ())),
                           preferred_element_type=jnp.float32)


def _dinv_block(dega_ref, degb_ref):
    deg_row = dega_ref[0] + degb_ref[0] + 1.0
    return lax.rsqrt(_colvec(deg_row, _R))


def _tc_scale_matmul(x, W0, dega, degb):
    def body(x_ref, w_ref, dega_ref, degb_ref, out_ref):
        dinv = _dinv_block(dega_ref, degb_ref)
        out_ref[...] = jnp.dot(x_ref[...], w_ref[...],
                               preferred_element_type=jnp.float32) * dinv

    return pl.pallas_call(
        body,
        grid=(_NB,),
        in_specs=[
            pl.BlockSpec((_R, _D), lambda i: (i, 0)),
            pl.BlockSpec((_D, _D), lambda i: (0, 0)),
            pl.BlockSpec((1, 1, _R), lambda i: (i, 0, 0)),
            pl.BlockSpec((1, 1, _R), lambda i: (i, 0, 0)),
        ],
        out_specs=pl.BlockSpec((_R, _D), lambda i: (i, 0)),
        out_shape=jax.ShapeDtypeStruct((_N, _D), jnp.float32),
    )(x, W0, dega, degb)


def _ln_relu(z, g_ref, be_ref):
    mu = jnp.mean(z, axis=-1, keepdims=True)
    d = z - mu
    var = jnp.mean(d * d, axis=-1, keepdims=True)
    zn = d * lax.rsqrt(var + 1e-5) * g_ref[...] + be_ref[...]
    return jnp.maximum(zn, 0.0)


def _tc_layer0(agg_a, agg_b, hs0, dega, degb, b0, g0, be0, W1):
    def body(a_ref, b_ref, hs_ref, dega_ref, degb_ref, bias_ref, g_ref,
             be_ref, w_ref, h0_ref, hs1_ref):
        dinv = _dinv_block(dega_ref, degb_ref)
        z = (a_ref[...] + b_ref[...] + hs_ref[...]) * dinv + bias_ref[...]
        h0 = _ln_relu(z, g_ref, be_ref)
        h0_ref[...] = h0
        hs1_ref[...] = jnp.dot(h0, w_ref[...],
                               preferred_element_type=jnp.float32) * dinv

    return pl.pallas_call(
        body,
        grid=(_NB,),
        in_specs=[
            pl.BlockSpec((_R, _D), lambda i: (i, 0)),
            pl.BlockSpec((_R, _D), lambda i: (i, 0)),
            pl.BlockSpec((_R, _D), lambda i: (i, 0)),
            pl.BlockSpec((1, 1, _R), lambda i: (i, 0, 0)),
            pl.BlockSpec((1, 1, _R), lambda i: (i, 0, 0)),
            pl.BlockSpec((1, _D), lambda i: (0, 0)),
            pl.BlockSpec((1, _D), lambda i: (0, 0)),
            pl.BlockSpec((1, _D), lambda i: (0, 0)),
            pl.BlockSpec((_D, _D), lambda i: (0, 0)),
        ],
        out_specs=[
            pl.BlockSpec((_R, _D), lambda i: (i, 0)),
            pl.BlockSpec((_R, _D), lambda i: (i, 0)),
        ],
        out_shape=[
            jax.ShapeDtypeStruct((_N, _D), jnp.float32),
            jax.ShapeDtypeStruct((_N, _D), jnp.float32),
        ],
    )(agg_a, agg_b, hs0, dega, degb, b0, g0, be0, W1)


def _tc_layer1_pool_head(agg_a, agg_b, hs1, h0, dega, degb, b1, g1, be1,
                         batch3, fW1, fb1, fW2, fb2, fW3, fb3):
    def body(a_ref, b_ref, hs_ref, h0_ref, dega_ref, degb_ref, bias_ref,
             g_ref, be_ref, bt_ref, w1_ref, c1_ref, w2_ref, c2_ref, w3_ref,
             c3_ref, out_ref, psum_v, pcnt_v):
        dinv = _dinv_block(dega_ref, degb_ref)
        z = (a_ref[...] + b_ref[...] + hs_ref[...]) * dinv + bias_ref[...]
        h1 = _ln_relu(z, g_ref, be_ref) + h0_ref[...]
        bt = bt_ref[0]
        gi = lax.broadcasted_iota(jnp.int32, (_G, _R), 0)
        oh = jnp.where(gi == bt, jnp.float32(1.0), jnp.float32(0.0))
        ps = jnp.dot(oh, h1, preferred_element_type=jnp.float32)
        cnt = jnp.sum(oh, axis=1, keepdims=True) * jnp.ones(
            (1, _D), jnp.float32)

        @pl.when(pl.program_id(0) == 0)
        def _():
            psum_v[...] = ps
            pcnt_v[...] = cnt

        @pl.when(pl.program_id(0) != 0)
        def _():
            psum_v[...] += ps
            pcnt_v[...] += cnt

        @pl.when(pl.program_id(0) == _NB - 1)
        def _():
            pooled = psum_v[...] / jnp.maximum(pcnt_v[...], 1.0)
            y = jnp.maximum(jnp.dot(pooled, w1_ref[...],
                                    preferred_element_type=jnp.float32)
                            + c1_ref[...], 0.0)
            y = jnp.maximum(jnp.dot(y, w2_ref[...],
                                    preferred_element_type=jnp.float32)
                            + c2_ref[...], 0.0)
            y = jnp.dot(y, w3_ref[...],
                        preferred_element_type=jnp.float32) + c3_ref[...]
            out_ref[...] = jnp.tanh(y)

    return pl.pallas_call(
        body,
        grid=(_NB,),
        in_specs=[
            pl.BlockSpec((_R, _D), lambda i: (i, 0)),
            pl.BlockSpec((_R, _D), lambda i: (i, 0)),
            pl.BlockSpec((_R, _D), lambda i: (i, 0)),
            pl.BlockSpec((_R, _D), lambda i: (i, 0)),
            pl.BlockSpec((1, 1, _R), lambda i: (i, 0, 0)),
            pl.BlockSpec((1, 1, _R), lambda i: (i, 0, 0)),
            pl.BlockSpec((1, _D), lambda i: (0, 0)),
            pl.BlockSpec((1, _D), lambda i: (0, 0)),
            pl.BlockSpec((1, _D), lambda i: (0, 0)),
            pl.BlockSpec((1, 1, _R), lambda i: (i, 0, 0)),
            pl.BlockSpec((_D, 2 * _D), lambda i: (0, 0)),
            pl.BlockSpec((1, 2 * _D), lambda i: (0, 0)),
            pl.BlockSpec((2 * _D, 2 * _D), lambda i: (0, 0)),
            pl.BlockSpec((1, 2 * _D), lambda i: (0, 0)),
            pl.BlockSpec((2 * _D, _D), lambda i: (0, 0)),
            pl.BlockSpec((1, _D), lambda i: (0, 0)),
        ],
        out_specs=pl.BlockSpec((_G, _D), lambda i: (0, 0)),
        out_shape=jax.ShapeDtypeStruct((_G, _D), jnp.float32),
        scratch_shapes=[
            pltpu.VMEM((_G, _D), jnp.float32),
            pltpu.VMEM((_G, _D), jnp.float32),
        ],
    )(agg_a, agg_b, hs1, h0, dega, degb, b1, g1, be1, batch3,
      fW1, fb1, fW2, fb2, fW3, fb3)


def kernel(x, edge_index, batch, W0, b0, g0, be0, W1, b1, g1, be1,
           fW1, fb1, fW2, fb2, fW3, fb3):
    src = edge_index[0].astype(jnp.int32).reshape(_NW, _SEGN, _SEG, _KE)
    dst = edge_index[1].astype(jnp.int32).reshape(_NW, _SEGN, _SEG, _KE)
    dst3 = dst.reshape(_NW, _NCH, _KE)

    degp = _sc_deg(dst3)
    dega = degp[0].reshape(_NB, 1, _R)
    degb = degp[1].reshape(_NB, 1, _R)

    hs0 = _tc_scale_matmul(x, W0, dega, degb)
    aggp0 = _sc_edge_agg(hs0, src, dst)
    h0, hs1 = _tc_layer0(aggp0[0], aggp0[1], hs0, dega, degb,
                         b0.reshape(1, _D), g0.reshape(1, _D),
                         be0.reshape(1, _D), W1)

    aggp1 = _sc_edge_agg(hs1, src, dst)
    batch3 = batch.astype(jnp.int32).reshape(_NB, 1, _R)
    return _tc_layer1_pool_head(aggp1[0], aggp1[1], hs1, h0, dega, degb,
                                b1.reshape(1, _D), g1.reshape(1, _D),
                                be1.reshape(1, _D), batch3,
                                fW1, fb1.reshape(1, -1),
                                fW2, fb2.reshape(1, -1),
                                fW3, fb3.reshape(1, -1))


# speedup vs baseline: 1.2572x; 1.0595x over previous
import functools

import jax
import jax.numpy as jnp
from jax import lax
from jax.experimental import pallas as pl
from jax.experimental.pallas import tpu as pltpu
from jax.experimental.pallas import tpu_sc as plsc

_N = 10000
_E = 320000
_D = 128
_G = 64

_NC = 2
_NS = 16
_NW = _NC * _NS
_EW = _E // _NW
_KE = 125
_NCH = _EW // _KE
_SEG = 16
_SEGN = _NCH // _SEG
_RZ = 104
_WT = 624
_WTAIL = _N - _NS * _WT

_R = 2000
_NB = _N // _R


def _sc_mesh():
    return plsc.VectorSubcoreMesh(core_axis_name="c", subcore_axis_name="s")


def _sc_deg(dst):
    @functools.partial(
        pl.kernel,
        out_type=jax.ShapeDtypeStruct((_NC, _N), jnp.float32),
        mesh=_sc_mesh(),
        scratch_types=[
            pltpu.VMEM((_NCH, _KE), jnp.int32),
            pltpu.VMEM((_KE,), jnp.float32),
            pltpu.VMEM((_N,), jnp.float32),
            pltpu.SemaphoreType.DMA,
            pltpu.VMEM_SHARED((_N,), jnp.float32),
        ],
    )
    def deg_kernel(dst_hbm, out_hbm, didx_v, ones_v, zero_v, sem, deg_sh):
        c = lax.axis_index("c")
        s = lax.axis_index("s")
        wid = c * _NS + s

        pltpu.sync_copy(dst_hbm.at[wid], didx_v)

        def fill_ones(i, carry):
            ones_v[pl.ds(i * 16, 16)] = jnp.ones((16,), jnp.float32)
            return carry

        lax.fori_loop(0, _KE // 16, fill_ones, 0)
        ones_v[pl.ds(_KE - 16, 16)] = jnp.ones((16,), jnp.float32)

        @pl.when(s == 0)
        def _():
            def fill_zero(i, carry):
                zero_v[pl.ds(i * 16, 16)] = jnp.zeros((16,), jnp.float32)
                return carry

            lax.fori_loop(0, _N // 16, fill_zero, 0)
            pltpu.sync_copy(zero_v, deg_sh)

        plsc.subcore_barrier()

        _GRP = 5

        def grp(g, carry):
            for t in range(_GRP):
                pltpu.async_copy(ones_v, deg_sh.at[didx_v.at[g * _GRP + t]],
                                 sem, add=True)
            for t in range(_GRP):
                pltpu.make_async_copy(ones_v, deg_sh.at[didx_v.at[0]],
                                      sem).wait()
            return carry

        lax.fori_loop(0, _NCH // _GRP, grp, 0)
        plsc.subcore_barrier()

        @pl.when(s == 0)
        def _():
            pltpu.sync_copy(deg_sh, out_hbm.at[c])

    return deg_kernel(dst)


def _sc_edge_agg(hs, src, dst):
    @functools.partial(
        pl.kernel,
        out_type=jax.ShapeDtypeStruct((_NC, _N, _D), jnp.float32),
        mesh=_sc_mesh(),
        scratch_types=[
            pltpu.VMEM((_SEG, _KE), jnp.int32),
            pltpu.VMEM((_SEG, _KE), jnp.int32),
            pltpu.VMEM((_KE, _D), jnp.float32),
            pltpu.VMEM((_KE, _D), jnp.float32),
            pltpu.SemaphoreType.DMA,
            pltpu.SemaphoreType.DMA,
            pltpu.VMEM_SHARED((_N, _D), jnp.float32),
        ],
    )
    def agg_kernel(hs_hbm, src_hbm, dst_hbm, out_hbm, sidx_v, didx_v,
                   rows0_v, rows1_v, sem0, sem1, agg_sh):
        c = lax.axis_index("c")
        s = lax.axis_index("s")
        wid = c * _NS + s

        def zrow(i, carry):
            for j in range(_D // 16):
                rows0_v[i, pl.ds(j * 16, 16)] = jnp.zeros((16,), jnp.float32)
            return carry

        lax.fori_loop(0, _KE, zrow, 0)

        def zchunk(k, carry):
            pltpu.sync_copy(rows0_v.at[pl.ds(0, _RZ)],
                            agg_sh.at[pl.ds(s * _WT + k * _RZ, _RZ)])
            return carry

        lax.fori_loop(0, _WT // _RZ, zchunk, 0)

        @pl.when(s == 0)
        def _():
            pltpu.sync_copy(rows0_v.at[pl.ds(0, _WTAIL)],
                            agg_sh.at[pl.ds(_NS * _WT, _WTAIL)])

        plsc.subcore_barrier()

        def seg_body(sg, carry):
            pltpu.sync_copy(src_hbm.at[wid, sg], sidx_v)
            pltpu.sync_copy(dst_hbm.at[wid, sg], didx_v)
            pltpu.async_copy(hs_hbm.at[sidx_v.at[0]], rows0_v, sem0)

            def step(g, carry2):
                j0 = 2 * g
                pltpu.async_copy(hs_hbm.at[sidx_v.at[j0 + 1]], rows1_v, sem1)
                pltpu.make_async_copy(hs_hbm.at[sidx_v.at[0]], rows0_v,
                                      sem0).wait()
                pltpu.sync_copy(rows0_v, agg_sh.at[didx_v.at[j0]], add=True)
                pltpu.async_copy(hs_hbm.at[sidx_v.at[j0 + 2]], rows0_v, sem0)
                pltpu.make_async_copy(hs_hbm.at[sidx_v.at[0]], rows1_v,
                                      sem1).wait()
                pltpu.sync_copy(rows1_v, agg_sh.at[didx_v.at[j0 + 1]],
                                add=True)
                return carry2

            lax.fori_loop(0, (_SEG - 2) // 2, step, 0)
            pltpu.async_copy(hs_hbm.at[sidx_v.at[_SEG - 1]], rows1_v, sem1)
            pltpu.make_async_copy(hs_hbm.at[sidx_v.at[0]], rows0_v,
                                  sem0).wait()
            pltpu.sync_copy(rows0_v, agg_sh.at[didx_v.at[_SEG - 2]], add=True)
            pltpu.make_async_copy(hs_hbm.at[sidx_v.at[0]], rows1_v,
                                  sem1).wait()
            pltpu.sync_copy(rows1_v, agg_sh.at[didx_v.at[_SEG - 1]], add=True)
            return carry

        lax.fori_loop(0, _SEGN, seg_body, 0)
        plsc.subcore_barrier()
        pltpu.sync_copy(agg_sh.at[pl.ds(s * _WT, _WT)],
                        out_hbm.at[c, pl.ds(s * _WT, _WT)])

        @pl.when(s == 0)
        def _():
            pltpu.sync_copy(agg_sh.at[pl.ds(_NS * _WT, _WTAIL)],
                            out_hbm.at[c, pl.ds(_NS * _WT, _WTAIL)])

    return agg_kernel(hs, src, dst)


def _dinv_block(dega_ref, degb_ref):
    return lax.rsqrt(dega_ref[...] + degb_ref[...] + 1.0)


def _tc_scale_matmul(x, W0, dega, degb):
    def body(x_ref, w_ref, dega_ref, degb_ref, out_ref):
        dinv = _dinv_block(dega_ref, degb_ref)
        out_ref[...] = jnp.dot(x_ref[...], w_ref[...],
                               preferred_element_type=jnp.float32) * dinv

    return pl.pallas_call(
        body,
        grid=(_NB,),
        in_specs=[
            pl.BlockSpec((_R, _D), lambda i: (i, 0)),
            pl.BlockSpec((_D, _D), lambda i: (0, 0)),
            pl.BlockSpec((_R, 1), lambda i: (i, 0)),
            pl.BlockSpec((_R, 1), lambda i: (i, 0)),
        ],
        out_specs=pl.BlockSpec((_R, _D), lambda i: (i, 0)),
        out_shape=jax.ShapeDtypeStruct((_N, _D), jnp.float32),
    )(x, W0, dega, degb)


def _ln_relu(z, g_ref, be_ref):
    mu = jnp.mean(z, axis=-1, keepdims=True)
    d = z - mu
    var = jnp.mean(d * d, axis=-1, keepdims=True)
    zn = d * lax.rsqrt(var + 1e-5) * g_ref[...] + be_ref[...]
    return jnp.maximum(zn, 0.0)


def _tc_layer0(agg_a, agg_b, hs0, dega, degb, b0, g0, be0, W1):
    def body(a_ref, b_ref, hs_ref, dega_ref, degb_ref, bias_ref, g_ref,
             be_ref, w_ref, h0_ref, hs1_ref):
        dinv = _dinv_block(dega_ref, degb_ref)
        z = (a_ref[...] + b_ref[...] + hs_ref[...]) * dinv + bias_ref[...]
        h0 = _ln_relu(z, g_ref, be_ref)
        h0_ref[...] = h0
        hs1_ref[...] = jnp.dot(h0, w_ref[...],
                               preferred_element_type=jnp.float32) * dinv

    return pl.pallas_call(
        body,
        grid=(_NB,),
        in_specs=[
            pl.BlockSpec((_R, _D), lambda i: (i, 0)),
            pl.BlockSpec((_R, _D), lambda i: (i, 0)),
            pl.BlockSpec((_R, _D), lambda i: (i, 0)),
            pl.BlockSpec((_R, 1), lambda i: (i, 0)),
            pl.BlockSpec((_R, 1), lambda i: (i, 0)),
            pl.BlockSpec((1, _D), lambda i: (0, 0)),
            pl.BlockSpec((1, _D), lambda i: (0, 0)),
            pl.BlockSpec((1, _D), lambda i: (0, 0)),
            pl.BlockSpec((_D, _D), lambda i: (0, 0)),
        ],
        out_specs=[
            pl.BlockSpec((_R, _D), lambda i: (i, 0)),
            pl.BlockSpec((_R, _D), lambda i: (i, 0)),
        ],
        out_shape=[
            jax.ShapeDtypeStruct((_N, _D), jnp.float32),
            jax.ShapeDtypeStruct((_N, _D), jnp.float32),
        ],
    )(agg_a, agg_b, hs0, dega, degb, b0, g0, be0, W1)


def _tc_layer1_pool_head(agg_a, agg_b, hs1, h0, dega, degb, b1, g1, be1,
                         batch3, fW1, fb1, fW2, fb2, fW3, fb3):
    def body(a_ref, b_ref, hs_ref, h0_ref, dega_ref, degb_ref, bias_ref,
             g_ref, be_ref, bt_ref, w1_ref, c1_ref, w2_ref, c2_ref, w3_ref,
             c3_ref, out_ref, psum_v, pcnt_v):
        dinv = _dinv_block(dega_ref, degb_ref)
        z = (a_ref[...] + b_ref[...] + hs_ref[...]) * dinv + bias_ref[...]
        h1 = _ln_relu(z, g_ref, be_ref) + h0_ref[...]
        bt = bt_ref[0]
        gi = lax.broadcasted_iota(jnp.int32, (_G, _R), 0)
        oh = jnp.where(gi == bt, jnp.float32(1.0), jnp.float32(0.0))
        ps = jnp.dot(oh, h1, preferred_element_type=jnp.float32)
        cnt = jnp.sum(oh, axis=1, keepdims=True) * jnp.ones(
            (1, _D), jnp.float32)

        @pl.when(pl.program_id(0) == 0)
        def _():
            psum_v[...] = ps
            pcnt_v[...] = cnt

        @pl.when(pl.program_id(0) != 0)
        def _():
            psum_v[...] += ps
            pcnt_v[...] += cnt

        @pl.when(pl.program_id(0) == _NB - 1)
        def _():
            pooled = psum_v[...] / jnp.maximum(pcnt_v[...], 1.0)
            y = jnp.maximum(jnp.dot(pooled, w1_ref[...],
                                    preferred_element_type=jnp.float32)
                            + c1_ref[...], 0.0)
            y = jnp.maximum(jnp.dot(y, w2_ref[...],
                                    preferred_element_type=jnp.float32)
                            + c2_ref[...], 0.0)
            y = jnp.dot(y, w3_ref[...],
                        preferred_element_type=jnp.float32) + c3_ref[...]
            out_ref[...] = jnp.tanh(y)

    return pl.pallas_call(
        body,
        grid=(_NB,),
        in_specs=[
            pl.BlockSpec((_R, _D), lambda i: (i, 0)),
            pl.BlockSpec((_R, _D), lambda i: (i, 0)),
            pl.BlockSpec((_R, _D), lambda i: (i, 0)),
            pl.BlockSpec((_R, _D), lambda i: (i, 0)),
            pl.BlockSpec((_R, 1), lambda i: (i, 0)),
            pl.BlockSpec((_R, 1), lambda i: (i, 0)),
            pl.BlockSpec((1, _D), lambda i: (0, 0)),
            pl.BlockSpec((1, _D), lambda i: (0, 0)),
            pl.BlockSpec((1, _D), lambda i: (0, 0)),
            pl.BlockSpec((1, 1, _R), lambda i: (i, 0, 0)),
            pl.BlockSpec((_D, 2 * _D), lambda i: (0, 0)),
            pl.BlockSpec((1, 2 * _D), lambda i: (0, 0)),
            pl.BlockSpec((2 * _D, 2 * _D), lambda i: (0, 0)),
            pl.BlockSpec((1, 2 * _D), lambda i: (0, 0)),
            pl.BlockSpec((2 * _D, _D), lambda i: (0, 0)),
            pl.BlockSpec((1, _D), lambda i: (0, 0)),
        ],
        out_specs=pl.BlockSpec((_G, _D), lambda i: (0, 0)),
        out_shape=jax.ShapeDtypeStruct((_G, _D), jnp.float32),
        scratch_shapes=[
            pltpu.VMEM((_G, _D), jnp.float32),
            pltpu.VMEM((_G, _D), jnp.float32),
        ],
    )(agg_a, agg_b, hs1, h0, dega, degb, b1, g1, be1, batch3,
      fW1, fb1, fW2, fb2, fW3, fb3)


def kernel(x, edge_index, batch, W0, b0, g0, be0, W1, b1, g1, be1,
           fW1, fb1, fW2, fb2, fW3, fb3):
    src = edge_index[0].astype(jnp.int32).reshape(_NW, _SEGN, _SEG, _KE)
    dst = edge_index[1].astype(jnp.int32).reshape(_NW, _SEGN, _SEG, _KE)
    dst3 = dst.reshape(_NW, _NCH, _KE)

    degp = _sc_deg(dst3)
    dega = degp[0].reshape(_N, 1)
    degb = degp[1].reshape(_N, 1)

    hs0 = _tc_scale_matmul(x, W0, dega, degb)
    aggp0 = _sc_edge_agg(hs0, src, dst)
    h0, hs1 = _tc_layer0(aggp0[0], aggp0[1], hs0, dega, degb,
                         b0.reshape(1, _D), g0.reshape(1, _D),
                         be0.reshape(1, _D), W1)

    aggp1 = _sc_edge_agg(hs1, src, dst)
    batch3 = batch.astype(jnp.int32).reshape(_NB, 1, _R)
    return _tc_layer1_pool_head(aggp1[0], aggp1[1], hs1, h0, dega, degb,
                                b1.reshape(1, _D), g1.reshape(1, _D),
                                be1.reshape(1, _D), batch3,
                                fW1, fb1.reshape(1, -1),
                                fW2, fb2.reshape(1, -1),
                                fW3, fb3.reshape(1, -1))


# speedup vs baseline: 1.3477x; 1.0720x over previous
import functools

import jax
import jax.numpy as jnp
from jax import lax
from jax.experimental import pallas as pl
from jax.experimental.pallas import tpu as pltpu
from jax.experimental.pallas import tpu_sc as plsc

_N = 10000
_E = 320000
_D = 128
_G = 64

_NC = 2
_NS = 16
_NW = _NC * _NS
_EW = _E // _NW
_KE = 125
_NCH = _EW // _KE
_SEG = 16
_SEGN = _NCH // _SEG
_RZ = 104
_WT = 624
_WTAIL = _N - _NS * _WT

_R = 2000
_NB = _N // _R


def _sc_mesh():
    return plsc.VectorSubcoreMesh(core_axis_name="c", subcore_axis_name="s")


def _sc_deg(ei):
    @functools.partial(
        pl.kernel,
        out_type=jax.ShapeDtypeStruct((_NC, _N), jnp.float32),
        mesh=_sc_mesh(),
        scratch_types=[
            pltpu.VMEM((_SEGN, _SEG, _KE), jnp.int32),
            pltpu.VMEM((_KE,), jnp.float32),
            pltpu.VMEM((_N,), jnp.float32),
            pltpu.SemaphoreType.DMA,
            pltpu.VMEM_SHARED((_N,), jnp.float32),
        ],
    )
    def deg_kernel(ei_hbm, out_hbm, didx_v, ones_v, zero_v, sem, deg_sh):
        c = lax.axis_index("c")
        s = lax.axis_index("s")
        wid = c * _NS + s

        pltpu.sync_copy(ei_hbm.at[1, wid], didx_v)

        def fill_ones(i, carry):
            ones_v[pl.ds(i * 16, 16)] = jnp.ones((16,), jnp.float32)
            return carry

        lax.fori_loop(0, _KE // 16, fill_ones, 0)
        ones_v[pl.ds(_KE - 16, 16)] = jnp.ones((16,), jnp.float32)

        @pl.when(s == 0)
        def _():
            def fill_zero(i, carry):
                zero_v[pl.ds(i * 16, 16)] = jnp.zeros((16,), jnp.float32)
                return carry

            lax.fori_loop(0, _N // 16, fill_zero, 0)
            pltpu.sync_copy(zero_v, deg_sh)

        plsc.subcore_barrier()

        _GRP = 4

        def grp(sg, carry):
            def grp2(gg, carry2):
                for t in range(_GRP):
                    pltpu.async_copy(
                        ones_v, deg_sh.at[didx_v.at[sg, gg * _GRP + t]],
                        sem, add=True)
                for t in range(_GRP):
                    pltpu.make_async_copy(ones_v, deg_sh.at[didx_v.at[0, 0]],
                                          sem).wait()
                return carry2

            lax.fori_loop(0, _SEG // _GRP, grp2, 0)
            return carry

        lax.fori_loop(0, _SEGN, grp, 0)
        plsc.subcore_barrier()

        @pl.when(s == 0)
        def _():
            pltpu.sync_copy(deg_sh, out_hbm.at[c])

    return deg_kernel(ei)


def _sc_edge_agg(hs, ei):
    @functools.partial(
        pl.kernel,
        out_type=jax.ShapeDtypeStruct((_NC, _N, _D), jnp.float32),
        mesh=_sc_mesh(),
        scratch_types=[
            pltpu.VMEM((_SEG, _KE), jnp.int32),
            pltpu.VMEM((_SEG, _KE), jnp.int32),
            pltpu.VMEM((_KE, _D), jnp.float32),
            pltpu.VMEM((_KE, _D), jnp.float32),
            pltpu.SemaphoreType.DMA,
            pltpu.SemaphoreType.DMA,
            pltpu.VMEM_SHARED((_N, _D), jnp.float32),
        ],
    )
    def agg_kernel(hs_hbm, ei_hbm, out_hbm, sidx_v, didx_v,
                   rows0_v, rows1_v, sem0, sem1, agg_sh):
        c = lax.axis_index("c")
        s = lax.axis_index("s")
        wid = c * _NS + s

        def zrow(i, carry):
            for j in range(_D // 16):
                rows0_v[i, pl.ds(j * 16, 16)] = jnp.zeros((16,), jnp.float32)
            return carry

        lax.fori_loop(0, _KE, zrow, 0)

        def zchunk(k, carry):
            pltpu.sync_copy(rows0_v.at[pl.ds(0, _RZ)],
                            agg_sh.at[pl.ds(s * _WT + k * _RZ, _RZ)])
            return carry

        lax.fori_loop(0, _WT // _RZ, zchunk, 0)

        @pl.when(s == 0)
        def _():
            pltpu.sync_copy(rows0_v.at[pl.ds(0, _WTAIL)],
                            agg_sh.at[pl.ds(_NS * _WT, _WTAIL)])

        plsc.subcore_barrier()

        def seg_body(sg, carry):
            pltpu.sync_copy(ei_hbm.at[0, wid, sg], sidx_v)
            pltpu.sync_copy(ei_hbm.at[1, wid, sg], didx_v)
            pltpu.async_copy(hs_hbm.at[sidx_v.at[0]], rows0_v, sem0)

            def step(g, carry2):
                j0 = 2 * g
                pltpu.async_copy(hs_hbm.at[sidx_v.at[j0 + 1]], rows1_v, sem1)
                pltpu.make_async_copy(hs_hbm.at[sidx_v.at[0]], rows0_v,
                                      sem0).wait()
                pltpu.sync_copy(rows0_v, agg_sh.at[didx_v.at[j0]], add=True)
                pltpu.async_copy(hs_hbm.at[sidx_v.at[j0 + 2]], rows0_v, sem0)
                pltpu.make_async_copy(hs_hbm.at[sidx_v.at[0]], rows1_v,
                                      sem1).wait()
                pltpu.sync_copy(rows1_v, agg_sh.at[didx_v.at[j0 + 1]],
                                add=True)
                return carry2

            lax.fori_loop(0, (_SEG - 2) // 2, step, 0)
            pltpu.async_copy(hs_hbm.at[sidx_v.at[_SEG - 1]], rows1_v, sem1)
            pltpu.make_async_copy(hs_hbm.at[sidx_v.at[0]], rows0_v,
                                  sem0).wait()
            pltpu.sync_copy(rows0_v, agg_sh.at[didx_v.at[_SEG - 2]], add=True)
            pltpu.make_async_copy(hs_hbm.at[sidx_v.at[0]], rows1_v,
                                  sem1).wait()
            pltpu.sync_copy(rows1_v, agg_sh.at[didx_v.at[_SEG - 1]], add=True)
            return carry

        lax.fori_loop(0, _SEGN, seg_body, 0)
        plsc.subcore_barrier()
        pltpu.sync_copy(agg_sh.at[pl.ds(s * _WT, _WT)],
                        out_hbm.at[c, pl.ds(s * _WT, _WT)])

        @pl.when(s == 0)
        def _():
            pltpu.sync_copy(agg_sh.at[pl.ds(_NS * _WT, _WTAIL)],
                            out_hbm.at[c, pl.ds(_NS * _WT, _WTAIL)])

    return agg_kernel(hs, ei)


def _dinv_block(dega_ref, degb_ref):
    return lax.rsqrt(dega_ref[0] + degb_ref[0] + 1.0)


_DEG_SPECS = [
    pl.BlockSpec((1, _R, 1), lambda i: (0, i, 0)),
    pl.BlockSpec((1, _R, 1), lambda i: (1, i, 0)),
]

_AGG_SPECS = [
    pl.BlockSpec((1, _R, _D), lambda i: (0, i, 0)),
    pl.BlockSpec((1, _R, _D), lambda i: (1, i, 0)),
]


def _tc_scale_matmul(x, W0, degp3):
    def body(x_ref, w_ref, dega_ref, degb_ref, out_ref):
        dinv = _dinv_block(dega_ref, degb_ref)
        out_ref[...] = jnp.dot(x_ref[...], w_ref[...],
                               preferred_element_type=jnp.float32) * dinv

    return pl.pallas_call(
        body,
        grid=(_NB,),
        in_specs=[
            pl.BlockSpec((_R, _D), lambda i: (i, 0)),
            pl.BlockSpec((_D, _D), lambda i: (0, 0)),
        ] + _DEG_SPECS,
        out_specs=pl.BlockSpec((_R, _D), lambda i: (i, 0)),
        out_shape=jax.ShapeDtypeStruct((_N, _D), jnp.float32),
    )(x, W0, degp3, degp3)


def _ln_relu(z, g_ref, be_ref):
    mu = jnp.mean(z, axis=-1, keepdims=True)
    d = z - mu
    var = jnp.mean(d * d, axis=-1, keepdims=True)
    zn = d * lax.rsqrt(var + 1e-5) * g_ref[...] + be_ref[...]
    return jnp.maximum(zn, 0.0)


def _tc_layer0(aggp, hs0, degp3, b0, g0, be0, W1):
    def body(a_ref, b_ref, hs_ref, dega_ref, degb_ref, bias_ref, g_ref,
             be_ref, w_ref, h0_ref, hs1_ref):
        dinv = _dinv_block(dega_ref, degb_ref)
        z = (a_ref[0] + b_ref[0] + hs_ref[...]) * dinv + bias_ref[...]
        h0 = _ln_relu(z, g_ref, be_ref)
        h0_ref[...] = h0
        hs1_ref[...] = jnp.dot(h0, w_ref[...],
                               preferred_element_type=jnp.float32) * dinv

    return pl.pallas_call(
        body,
        grid=(_NB,),
        in_specs=_AGG_SPECS + [
            pl.BlockSpec((_R, _D), lambda i: (i, 0)),
        ] + _DEG_SPECS + [
            pl.BlockSpec((1, _D), lambda i: (0, 0)),
            pl.BlockSpec((1, _D), lambda i: (0, 0)),
            pl.BlockSpec((1, _D), lambda i: (0, 0)),
            pl.BlockSpec((_D, _D), lambda i: (0, 0)),
        ],
        out_specs=[
            pl.BlockSpec((_R, _D), lambda i: (i, 0)),
            pl.BlockSpec((_R, _D), lambda i: (i, 0)),
        ],
        out_shape=[
            jax.ShapeDtypeStruct((_N, _D), jnp.float32),
            jax.ShapeDtypeStruct((_N, _D), jnp.float32),
        ],
    )(aggp, aggp, hs0, degp3, degp3, b0, g0, be0, W1)


def _tc_layer1_pool_head(aggp, hs1, h0, degp3, b1, g1, be1,
                         batch3, fW1, fb1, fW2, fb2, fW3, fb3):
    def body(a_ref, b_ref, hs_ref, h0_ref, dega_ref, degb_ref, bias_ref,
             g_ref, be_ref, bt_ref, w1_ref, c1_ref, w2_ref, c2_ref, w3_ref,
             c3_ref, out_ref, psum_v, pcnt_v):
        dinv = _dinv_block(dega_ref, degb_ref)
        z = (a_ref[0] + b_ref[0] + hs_ref[...]) * dinv + bias_ref[...]
        h1 = _ln_relu(z, g_ref, be_ref) + h0_ref[...]
        bt = bt_ref[0]
        gi = lax.broadcasted_iota(jnp.int32, (_G, _R), 0)
        oh = jnp.where(gi == bt, jnp.float32(1.0), jnp.float32(0.0))
        ps = jnp.dot(oh, h1, preferred_element_type=jnp.float32)
        cnt = jnp.sum(oh, axis=1, keepdims=True) * jnp.ones(
            (1, _D), jnp.float32)

        @pl.when(pl.program_id(0) == 0)
        def _():
            psum_v[...] = ps
            pcnt_v[...] = cnt

        @pl.when(pl.program_id(0) != 0)
        def _():
            psum_v[...] += ps
            pcnt_v[...] += cnt

        @pl.when(pl.program_id(0) == _NB - 1)
        def _():
            pooled = psum_v[...] / jnp.maximum(pcnt_v[...], 1.0)
            y = jnp.maximum(jnp.dot(pooled, w1_ref[...],
                                    preferred_element_type=jnp.float32)
                            + c1_ref[...], 0.0)
            y = jnp.maximum(jnp.dot(y, w2_ref[...],
                                    preferred_element_type=jnp.float32)
                            + c2_ref[...], 0.0)
            y = jnp.dot(y, w3_ref[...],
                        preferred_element_type=jnp.float32) + c3_ref[...]
            out_ref[...] = jnp.tanh(y)

    return pl.pallas_call(
        body,
        grid=(_NB,),
        in_specs=_AGG_SPECS + [
            pl.BlockSpec((_R, _D), lambda i: (i, 0)),
            pl.BlockSpec((_R, _D), lambda i: (i, 0)),
        ] + _DEG_SPECS + [
            pl.BlockSpec((1, _D), lambda i: (0, 0)),
            pl.BlockSpec((1, _D), lambda i: (0, 0)),
            pl.BlockSpec((1, _D), lambda i: (0, 0)),
            pl.BlockSpec((1, 1, _R), lambda i: (i, 0, 0)),
            pl.BlockSpec((_D, 2 * _D), lambda i: (0, 0)),
            pl.BlockSpec((1, 2 * _D), lambda i: (0, 0)),
            pl.BlockSpec((2 * _D, 2 * _D), lambda i: (0, 0)),
            pl.BlockSpec((1, 2 * _D), lambda i: (0, 0)),
            pl.BlockSpec((2 * _D, _D), lambda i: (0, 0)),
            pl.BlockSpec((1, _D), lambda i: (0, 0)),
        ],
        out_specs=pl.BlockSpec((_G, _D), lambda i: (0, 0)),
        out_shape=jax.ShapeDtypeStruct((_G, _D), jnp.float32),
        scratch_shapes=[
            pltpu.VMEM((_G, _D), jnp.float32),
            pltpu.VMEM((_G, _D), jnp.float32),
        ],
    )(aggp, aggp, hs1, h0, degp3, degp3, b1, g1, be1, batch3,
      fW1, fb1, fW2, fb2, fW3, fb3)


def kernel(x, edge_index, batch, W0, b0, g0, be0, W1, b1, g1, be1,
           fW1, fb1, fW2, fb2, fW3, fb3):
    ei = edge_index.astype(jnp.int32).reshape(2, _NW, _SEGN, _SEG, _KE)

    degp3 = _sc_deg(ei).reshape(_NC, _N, 1)

    hs0 = _tc_scale_matmul(x, W0, degp3)
    aggp0 = _sc_edge_agg(hs0, ei)
    h0, hs1 = _tc_layer0(aggp0, hs0, degp3,
                         b0.reshape(1, _D), g0.reshape(1, _D),
                         be0.reshape(1, _D), W1)

    aggp1 = _sc_edge_agg(hs1, ei)
    batch3 = batch.astype(jnp.int32).reshape(_NB, 1, _R)
    return _tc_layer1_pool_head(aggp1, hs1, h0, degp3,
                                b1.reshape(1, _D), g1.reshape(1, _D),
                                be1.reshape(1, _D), batch3,
                                fW1, fb1.reshape(1, -1),
                                fW2, fb2.reshape(1, -1),
                                fW3, fb3.reshape(1, -1))


# speedup vs baseline: 1.3522x; 1.0033x over previous
import functools

import jax
import jax.numpy as jnp
from jax import lax
from jax.experimental import pallas as pl
from jax.experimental.pallas import tpu as pltpu
from jax.experimental.pallas import tpu_sc as plsc

_N = 10000
_E = 320000
_D = 128
_G = 64

_NC = 2
_NS = 16
_NW = _NC * _NS
_EW = _E // _NW
_KE = 125
_NCH = _EW // _KE
_SEG = 16
_SEGN = _NCH // _SEG
_RZ = 104
_WT = 624
_WTAIL = _N - _NS * _WT

_R = 2000
_NB = _N // _R


def _sc_mesh():
    return plsc.VectorSubcoreMesh(core_axis_name="c", subcore_axis_name="s")


def _sc_deg(ei):
    @functools.partial(
        pl.kernel,
        out_type=jax.ShapeDtypeStruct((_NC, _N), jnp.float32),
        mesh=_sc_mesh(),
        scratch_types=[
            pltpu.VMEM((_SEGN, _SEG, _KE), jnp.int32),
            pltpu.VMEM((_KE,), jnp.float32),
            pltpu.VMEM((_N,), jnp.float32),
            pltpu.SemaphoreType.DMA,
            pltpu.VMEM_SHARED((_N,), jnp.float32),
        ],
    )
    def deg_kernel(ei_hbm, out_hbm, didx_v, ones_v, zero_v, sem, deg_sh):
        c = lax.axis_index("c")
        s = lax.axis_index("s")
        wid = c * _NS + s

        pltpu.sync_copy(ei_hbm.at[1, wid], didx_v)

        def fill_ones(i, carry):
            ones_v[pl.ds(i * 16, 16)] = jnp.ones((16,), jnp.float32)
            return carry

        lax.fori_loop(0, _KE // 16, fill_ones, 0)
        ones_v[pl.ds(_KE - 16, 16)] = jnp.ones((16,), jnp.float32)

        @pl.when(s == 0)
        def _():
            def fill_zero(i, carry):
                zero_v[pl.ds(i * 16, 16)] = jnp.zeros((16,), jnp.float32)
                return carry

            lax.fori_loop(0, _N // 16, fill_zero, 0)
            pltpu.sync_copy(zero_v, deg_sh)

        plsc.subcore_barrier()

        _GRP = 8

        def grp(sg, carry):
            def grp2(gg, carry2):
                for t in range(_GRP):
                    pltpu.async_copy(
                        ones_v, deg_sh.at[didx_v.at[sg, gg * _GRP + t]],
                        sem, add=True)
                for t in range(_GRP):
                    pltpu.make_async_copy(ones_v, deg_sh.at[didx_v.at[0, 0]],
                                          sem).wait()
                return carry2

            lax.fori_loop(0, _SEG // _GRP, grp2, 0)
            return carry

        lax.fori_loop(0, _SEGN, grp, 0)
        plsc.subcore_barrier()

        @pl.when(s == 0)
        def _():
            pltpu.sync_copy(deg_sh, out_hbm.at[c])

    return deg_kernel(ei)


def _sc_edge_agg(hs, ei):
    @functools.partial(
        pl.kernel,
        out_type=jax.ShapeDtypeStruct((_NC, _N, _D), jnp.float32),
        mesh=_sc_mesh(),
        scratch_types=[
            pltpu.VMEM((_SEG, _KE), jnp.int32),
            pltpu.VMEM((_SEG, _KE), jnp.int32),
            pltpu.VMEM((_KE, _D), jnp.float32),
            pltpu.VMEM((_KE, _D), jnp.float32),
            pltpu.SemaphoreType.DMA,
            pltpu.SemaphoreType.DMA,
            pltpu.VMEM_SHARED((_N, _D), jnp.float32),
        ],
    )
    def agg_kernel(hs_hbm, ei_hbm, out_hbm, sidx_v, didx_v,
                   rows0_v, rows1_v, sem0, sem1, agg_sh):
        c = lax.axis_index("c")
        s = lax.axis_index("s")
        wid = c * _NS + s

        def zrow(i, carry):
            for j in range(_D // 16):
                rows0_v[i, pl.ds(j * 16, 16)] = jnp.zeros((16,), jnp.float32)
            return carry

        lax.fori_loop(0, _KE, zrow, 0)

        def zchunk(k, carry):
            pltpu.sync_copy(rows0_v.at[pl.ds(0, _RZ)],
                            agg_sh.at[pl.ds(s * _WT + k * _RZ, _RZ)])
            return carry

        lax.fori_loop(0, _WT // _RZ, zchunk, 0)

        @pl.when(s == 0)
        def _():
            pltpu.sync_copy(rows0_v.at[pl.ds(0, _WTAIL)],
                            agg_sh.at[pl.ds(_NS * _WT, _WTAIL)])

        plsc.subcore_barrier()

        def seg_body(sg, carry):
            pltpu.sync_copy(ei_hbm.at[0, wid, sg], sidx_v)
            pltpu.sync_copy(ei_hbm.at[1, wid, sg], didx_v)
            pltpu.async_copy(hs_hbm.at[sidx_v.at[0]], rows0_v, sem0)

            def step(g, carry2):
                j0 = 2 * g
                pltpu.async_copy(hs_hbm.at[sidx_v.at[j0 + 1]], rows1_v, sem1)
                pltpu.make_async_copy(hs_hbm.at[sidx_v.at[0]], rows0_v,
                                      sem0).wait()
                pltpu.sync_copy(rows0_v, agg_sh.at[didx_v.at[j0]], add=True)
                pltpu.async_copy(hs_hbm.at[sidx_v.at[j0 + 2]], rows0_v, sem0)
                pltpu.make_async_copy(hs_hbm.at[sidx_v.at[0]], rows1_v,
                                      sem1).wait()
                pltpu.sync_copy(rows1_v, agg_sh.at[didx_v.at[j0 + 1]],
                                add=True)
                return carry2

            lax.fori_loop(0, (_SEG - 2) // 2, step, 0)
            pltpu.async_copy(hs_hbm.at[sidx_v.at[_SEG - 1]], rows1_v, sem1)
            pltpu.make_async_copy(hs_hbm.at[sidx_v.at[0]], rows0_v,
                                  sem0).wait()
            pltpu.sync_copy(rows0_v, agg_sh.at[didx_v.at[_SEG - 2]], add=True)
            pltpu.make_async_copy(hs_hbm.at[sidx_v.at[0]], rows1_v,
                                  sem1).wait()
            pltpu.sync_copy(rows1_v, agg_sh.at[didx_v.at[_SEG - 1]], add=True)
            return carry

        lax.fori_loop(0, _SEGN, seg_body, 0)
        plsc.subcore_barrier()
        pltpu.sync_copy(agg_sh.at[pl.ds(s * _WT, _WT)],
                        out_hbm.at[c, pl.ds(s * _WT, _WT)])

        @pl.when(s == 0)
        def _():
            pltpu.sync_copy(agg_sh.at[pl.ds(_NS * _WT, _WTAIL)],
                            out_hbm.at[c, pl.ds(_NS * _WT, _WTAIL)])

    return agg_kernel(hs, ei)


def _dinv_block(dega_ref, degb_ref):
    return lax.rsqrt(dega_ref[0] + degb_ref[0] + 1.0)


_DEG_SPECS = [
    pl.BlockSpec((1, _R, 1), lambda i: (0, i, 0)),
    pl.BlockSpec((1, _R, 1), lambda i: (1, i, 0)),
]

_AGG_SPECS = [
    pl.BlockSpec((1, _R, _D), lambda i: (0, i, 0)),
    pl.BlockSpec((1, _R, _D), lambda i: (1, i, 0)),
]


def _tc_scale_matmul(x, W0, degp3):
    def body(x_ref, w_ref, dega_ref, degb_ref, out_ref):
        dinv = _dinv_block(dega_ref, degb_ref)
        out_ref[...] = jnp.dot(x_ref[...], w_ref[...],
                               preferred_element_type=jnp.float32) * dinv

    return pl.pallas_call(
        body,
        grid=(_NB,),
        in_specs=[
            pl.BlockSpec((_R, _D), lambda i: (i, 0)),
            pl.BlockSpec((_D, _D), lambda i: (0, 0)),
        ] + _DEG_SPECS,
        out_specs=pl.BlockSpec((_R, _D), lambda i: (i, 0)),
        out_shape=jax.ShapeDtypeStruct((_N, _D), jnp.float32),
    )(x, W0, degp3, degp3)


def _ln_relu(z, g_ref, be_ref):
    mu = jnp.mean(z, axis=-1, keepdims=True)
    d = z - mu
    var = jnp.mean(d * d, axis=-1, keepdims=True)
    zn = d * lax.rsqrt(var + 1e-5) * g_ref[...] + be_ref[...]
    return jnp.maximum(zn, 0.0)


def _tc_layer0(aggp, hs0, degp3, b0, g0, be0, W1):
    def body(a_ref, b_ref, hs_ref, dega_ref, degb_ref, bias_ref, g_ref,
             be_ref, w_ref, h0_ref, hs1_ref):
        dinv = _dinv_block(dega_ref, degb_ref)
        z = (a_ref[0] + b_ref[0] + hs_ref[...]) * dinv + bias_ref[...]
        h0 = _ln_relu(z, g_ref, be_ref)
        h0_ref[...] = h0
        hs1_ref[...] = jnp.dot(h0, w_ref[...],
                               preferred_element_type=jnp.float32) * dinv

    return pl.pallas_call(
        body,
        grid=(_NB,),
        in_specs=_AGG_SPECS + [
            pl.BlockSpec((_R, _D), lambda i: (i, 0)),
        ] + _DEG_SPECS + [
            pl.BlockSpec((1, _D), lambda i: (0, 0)),
            pl.BlockSpec((1, _D), lambda i: (0, 0)),
            pl.BlockSpec((1, _D), lambda i: (0, 0)),
            pl.BlockSpec((_D, _D), lambda i: (0, 0)),
        ],
        out_specs=[
            pl.BlockSpec((_R, _D), lambda i: (i, 0)),
            pl.BlockSpec((_R, _D), lambda i: (i, 0)),
        ],
        out_shape=[
            jax.ShapeDtypeStruct((_N, _D), jnp.float32),
            jax.ShapeDtypeStruct((_N, _D), jnp.float32),
        ],
    )(aggp, aggp, hs0, degp3, degp3, b0, g0, be0, W1)


def _tc_layer1_pool_head(aggp, hs1, h0, degp3, b1, g1, be1,
                         batch3, fW1, fb1, fW2, fb2, fW3, fb3):
    def body(a_ref, b_ref, hs_ref, h0_ref, dega_ref, degb_ref, bias_ref,
             g_ref, be_ref, bt_ref, w1_ref, c1_ref, w2_ref, c2_ref, w3_ref,
             c3_ref, out_ref, psum_v, pcnt_v):
        dinv = _dinv_block(dega_ref, degb_ref)
        z = (a_ref[0] + b_ref[0] + hs_ref[...]) * dinv + bias_ref[...]
        h1 = _ln_relu(z, g_ref, be_ref) + h0_ref[...]
        bt = bt_ref[0]
        gi = lax.broadcasted_iota(jnp.int32, (_G, _R), 0)
        oh = jnp.where(gi == bt, jnp.float32(1.0), jnp.float32(0.0))
        ps = jnp.dot(oh, h1, preferred_element_type=jnp.float32)
        cnt = jnp.sum(oh, axis=1, keepdims=True) * jnp.ones(
            (1, _D), jnp.float32)

        @pl.when(pl.program_id(0) == 0)
        def _():
            psum_v[...] = ps
            pcnt_v[...] = cnt

        @pl.when(pl.program_id(0) != 0)
        def _():
            psum_v[...] += ps
            pcnt_v[...] += cnt

        @pl.when(pl.program_id(0) == _NB - 1)
        def _():
            pooled = psum_v[...] / jnp.maximum(pcnt_v[...], 1.0)
            y = jnp.maximum(jnp.dot(pooled, w1_ref[...],
                                    preferred_element_type=jnp.float32)
                            + c1_ref[...], 0.0)
            y = jnp.maximum(jnp.dot(y, w2_ref[...],
                                    preferred_element_type=jnp.float32)
                            + c2_ref[...], 0.0)
            y = jnp.dot(y, w3_ref[...],
                        preferred_element_type=jnp.float32) + c3_ref[...]
            out_ref[...] = jnp.tanh(y)

    return pl.pallas_call(
        body,
        grid=(_NB,),
        in_specs=_AGG_SPECS + [
            pl.BlockSpec((_R, _D), lambda i: (i, 0)),
            pl.BlockSpec((_R, _D), lambda i: (i, 0)),
        ] + _DEG_SPECS + [
            pl.BlockSpec((1, _D), lambda i: (0, 0)),
            pl.BlockSpec((1, _D), lambda i: (0, 0)),
            pl.BlockSpec((1, _D), lambda i: (0, 0)),
            pl.BlockSpec((1, 1, _R), lambda i: (i, 0, 0)),
            pl.BlockSpec((_D, 2 * _D), lambda i: (0, 0)),
            pl.BlockSpec((1, 2 * _D), lambda i: (0, 0)),
            pl.BlockSpec((2 * _D, 2 * _D), lambda i: (0, 0)),
            pl.BlockSpec((1, 2 * _D), lambda i: (0, 0)),
            pl.BlockSpec((2 * _D, _D), lambda i: (0, 0)),
            pl.BlockSpec((1, _D), lambda i: (0, 0)),
        ],
        out_specs=pl.BlockSpec((_G, _D), lambda i: (0, 0)),
        out_shape=jax.ShapeDtypeStruct((_G, _D), jnp.float32),
        scratch_shapes=[
            pltpu.VMEM((_G, _D), jnp.float32),
            pltpu.VMEM((_G, _D), jnp.float32),
        ],
    )(aggp, aggp, hs1, h0, degp3, degp3, b1, g1, be1, batch3,
      fW1, fb1, fW2, fb2, fW3, fb3)


def kernel(x, edge_index, batch, W0, b0, g0, be0, W1, b1, g1, be1,
           fW1, fb1, fW2, fb2, fW3, fb3):
    ei = edge_index.astype(jnp.int32).reshape(2, _NW, _SEGN, _SEG, _KE)

    degp3 = _sc_deg(ei).reshape(_NC, _N, 1)

    hs0 = _tc_scale_matmul(x, W0, degp3)
    aggp0 = _sc_edge_agg(hs0, ei)
    h0, hs1 = _tc_layer0(aggp0, hs0, degp3,
                         b0.reshape(1, _D), g0.reshape(1, _D),
                         be0.reshape(1, _D), W1)

    aggp1 = _sc_edge_agg(hs1, ei)
    batch3 = batch.astype(jnp.int32).reshape(_NB, 1, _R)
    return _tc_layer1_pool_head(aggp1, hs1, h0, degp3,
                                b1.reshape(1, _D), g1.reshape(1, _D),
                                be1.reshape(1, _D), batch3,
                                fW1, fb1.reshape(1, -1),
                                fW2, fb2.reshape(1, -1),
                                fW3, fb3.reshape(1, -1))
